# baseline (speedup 1.0000x reference)
import functools

import jax
import jax.numpy as jnp
from jax import lax
from jax.experimental import pallas as pl
from jax.experimental.pallas import tpu as pltpu
from jax.experimental.pallas import tpu_sc as plsc

VOCAB = 100000
EMB = 64
NPROP = 26
BATCH = 16384

NC = 2
NS = 16
NW = NC * NS
B_W = BATCH // NW
ICHUNK = 128
NJ = B_W // ICHUNK
LANES = 16


def _emb_body(propt_hbm, table_hbm, out_hbm, idx_v, rows_v, acc_v, sem):
    wid = lax.axis_index("s") * NC + lax.axis_index("c")
    base = wid * B_W

    zeros = jnp.zeros((LANES,), jnp.float32)

    def zero_step(i, _):
        for c in range(EMB // LANES):
            acc_v[i, pl.ds(c * LANES, LANES)] = zeros
        return 0

    lax.fori_loop(0, B_W, zero_step, 0)

    def p_step(p, _):
        for j in range(NJ):
            pltpu.sync_copy(
                propt_hbm.at[pl.ds(p * BATCH + base + j * ICHUNK, ICHUNK)],
                idx_v.at[j],
            )
        off = p * jnp.int32(VOCAB)
        for j in range(NJ):
            for c in range(ICHUNK // LANES):
                sl = pl.ds(c * LANES, LANES)
                idx_v[j, sl] = idx_v[j, sl] + off
        cps = [
            pltpu.async_copy(
                table_hbm.at[idx_v.at[j]],
                rows_v.at[pl.ds(j * ICHUNK, ICHUNK)],
                sem,
            )
            for j in range(NJ)
        ]
        for cp in cps:
            cp.wait()

        def acc_step(i, _):
            for c in range(EMB // LANES):
                sl = pl.ds(c * LANES, LANES)
                plsc.addupdate(acc_v.at[i, sl], rows_v[i, sl])
            return 0

        lax.fori_loop(0, B_W, acc_step, 0)
        return 0

    lax.fori_loop(0, NPROP, p_step, 0)

    pltpu.sync_copy(acc_v, out_hbm.at[pl.ds(base, B_W)])


@jax.jit
def _emb_call(propt3, flat_table):
    mesh = plsc.VectorSubcoreMesh(core_axis_name="c", subcore_axis_name="s")
    f = functools.partial(
        pl.kernel,
        out_type=jax.ShapeDtypeStruct((BATCH, EMB), jnp.float32),
        mesh=mesh,
        scratch_types=[
            pltpu.VMEM((NJ, ICHUNK), jnp.int32),
            pltpu.VMEM((B_W, EMB), jnp.float32),
            pltpu.VMEM((B_W, EMB), jnp.float32),
            pltpu.SemaphoreType.DMA,
        ],
        compiler_params=pltpu.CompilerParams(use_tc_tiling_on_sc=False),
    )(_emb_body)
    return f(propt3, flat_table)


def kernel(prop, tables):
    propt_flat = prop.astype(jnp.int32).T.reshape(-1)
    flat_table = tables.reshape(NPROP * VOCAB, EMB)
    out = _emb_call(propt_flat, flat_table)
    return out[:, None, :]


# speedup vs baseline: 1.0611x; 1.0611x over previous
import functools

import jax
import jax.numpy as jnp
from jax import lax
from jax.experimental import pallas as pl
from jax.experimental.pallas import tpu as pltpu
from jax.experimental.pallas import tpu_sc as plsc

VOCAB = 100000
EMB = 64
NPROP = 26
BATCH = 16384

NC = 2
NS = 16
NW = NC * NS
B_W = BATCH // NW
ICHUNK = 128
NJ = B_W // ICHUNK
IDXN = NPROP * B_W
LANES = 16
ROW_BYTES = EMB * 4


def _fire(table_hbm, idx_v, buf, sem, p):
    for j in range(NJ):
        pltpu.async_copy(
            table_hbm.at[idx_v.at[pl.ds((p * NJ + j) * ICHUNK, ICHUNK)]],
            buf.at[pl.ds(j * ICHUNK, ICHUNK)],
            sem,
        )


def _drain(table_hbm, buf, sem):
    pltpu.make_async_copy(table_hbm.at[pl.ds(0, B_W)], buf, sem).wait()


def _accumulate(acc_v, buf):
    @plsc.parallel_loop(0, B_W, unroll=8)
    def _(i):
        for c in range(EMB // LANES):
            sl = pl.ds(c * LANES, LANES)
            plsc.addupdate(acc_v.at[i, sl], buf[i, sl])


def _emb_body(propw_hbm, table_hbm, out_hbm, idx_v, buf_a, buf_b, acc_v,
              sem_a, sem_b):
    wid = lax.axis_index("s") * NC + lax.axis_index("c")
    base = wid * B_W

    pltpu.sync_copy(propw_hbm.at[pl.ds(wid * IDXN, IDXN)], idx_v)

    @plsc.parallel_loop(0, IDXN // LANES, unroll=4)
    def _(k):
        off = (k // (B_W // LANES)) * jnp.int32(VOCAB)
        sl = pl.ds(k * LANES, LANES)
        idx_v[sl] = idx_v[sl] + off

    zeros = jnp.zeros((LANES,), jnp.float32)

    @plsc.parallel_loop(0, B_W, unroll=8)
    def _(i):
        for c in range(EMB // LANES):
            acc_v[i, pl.ds(c * LANES, LANES)] = zeros

    _fire(table_hbm, idx_v, buf_a, sem_a, jnp.int32(0))

    def pp_step(pp, _):
        p = 2 * pp
        _fire(table_hbm, idx_v, buf_b, sem_b, p + 1)
        _drain(table_hbm, buf_a, sem_a)
        _accumulate(acc_v, buf_a)

        @pl.when(pp < NPROP // 2 - 1)
        def _():
            _fire(table_hbm, idx_v, buf_a, sem_a, p + 2)

        _drain(table_hbm, buf_b, sem_b)
        _accumulate(acc_v, buf_b)
        return 0

    lax.fori_loop(0, NPROP // 2, pp_step, 0)

    pltpu.sync_copy(acc_v, out_hbm.at[pl.ds(base, B_W)])


@jax.jit
def _emb_call(propw, flat_table):
    mesh = plsc.VectorSubcoreMesh(core_axis_name="c", subcore_axis_name="s")
    f = functools.partial(
        pl.kernel,
        out_type=jax.ShapeDtypeStruct((BATCH, EMB), jnp.float32),
        mesh=mesh,
        scratch_types=[
            pltpu.VMEM((IDXN,), jnp.int32),
            pltpu.VMEM((B_W, EMB), jnp.float32),
            pltpu.VMEM((B_W, EMB), jnp.float32),
            pltpu.VMEM((B_W, EMB), jnp.float32),
            pltpu.SemaphoreType.DMA,
            pltpu.SemaphoreType.DMA,
        ],
        compiler_params=pltpu.CompilerParams(use_tc_tiling_on_sc=False),
    )(_emb_body)
    return f(propw, flat_table)


def kernel(prop, tables):
    propw = (
        prop.astype(jnp.int32)
        .reshape(NW, B_W, NPROP)
        .transpose(0, 2, 1)
        .reshape(-1)
    )
    flat_table = tables.reshape(NPROP * VOCAB, EMB)
    out = _emb_call(propw, flat_table)
    return out[:, None, :]
